# Initial kernel scaffold; baseline (speedup 1.0000x reference)
import jax
import jax.numpy as jnp
from jax.experimental import pallas as pl


def kernel(x, edge_index, W1, b1, g1, be1, W2, b2, g2, be2, W3, b3, g3, be3, Wl1, bl1, g4, be4, Wl2, bl2):
    raise NotImplementedError("write your pallas kernel here")


# speedup vs baseline: 1.3485x; 1.3485x over previous
import jax
import jax.numpy as jnp
from jax.experimental import pallas as pl
from jax.experimental.pallas import tpu as pltpu


def _head_kernel(h_ref, wl1_ref, bl1_ref, g4_ref, be4_ref, wl2_ref, bl2_ref, out_ref):
    h = h_ref[...]
    y = jnp.dot(h, wl1_ref[...], preferred_element_type=jnp.float32) + bl1_ref[...]
    mu = jnp.mean(y, axis=0, keepdims=True)
    var = jnp.mean((y - mu) ** 2, axis=0, keepdims=True)
    y = (y - mu) / jnp.sqrt(var + 1e-5) * g4_ref[...] + be4_ref[...]
    y = jnp.maximum(y, 0.0)
    out_ref[...] = jnp.dot(y, wl2_ref[...], preferred_element_type=jnp.float32) + bl2_ref[...]


def _head(h, Wl1, bl1, g4, be4, Wl2, bl2):
    n = h.shape[0]
    return pl.pallas_call(
        _head_kernel,
        out_shape=jax.ShapeDtypeStruct((n, 1), jnp.float32),
    )(h, Wl1.T, bl1[None, :], g4[None, :], be4[None, :], Wl2.T, bl2[None, :])


def _bn_relu(x, gamma, beta, eps=1e-5):
    mu = jnp.mean(x, axis=0)
    var = jnp.var(x, axis=0)
    return jax.nn.relu((x - mu) / jnp.sqrt(var + eps) * gamma + beta)


def _conv(x, src, dst, cnt, W, b):
    f = x.shape[1]
    Wd, Ws = W[:, :f], W[:, f:]
    yd = x @ Wd.T + b
    ys = x @ Ws.T
    ys_e = jnp.take(ys, src, axis=0)
    n = x.shape[0]
    S = jax.ops.segment_sum(ys_e, dst, num_segments=n)
    M = jax.ops.segment_max(ys_e, dst, num_segments=n)
    has = cnt[:, None] > 0
    s = cnt[:, None] * yd + S
    mean = jnp.where(has, yd + S / jnp.maximum(cnt, 1.0)[:, None], 0.0)
    mx = jnp.where(has, yd + M, 0.0)
    return jnp.concatenate([s, mean, mx], axis=1)


def kernel(x, edge_index, W1, b1, g1, be1, W2, b2, g2, be2, W3, b3, g3, be3, Wl1, bl1, g4, be4, Wl2, bl2):
    src = edge_index[0]
    dst = edge_index[1]
    n = x.shape[0]
    cnt = jax.ops.segment_sum(jnp.ones_like(dst, jnp.float32), dst, num_segments=n)
    h = _bn_relu(_conv(x, src, dst, cnt, W1, b1), g1, be1)
    h = _bn_relu(_conv(h, src, dst, cnt, W2, b2), g2, be2)
    h = _bn_relu(_conv(h, src, dst, cnt, W3, b3), g3, be3)
    return _head(h, Wl1, bl1, g4, be4, Wl2, bl2)


# speedup vs baseline: 11.5851x; 8.5909x over previous
import functools

import jax
import jax.numpy as jnp
from jax import lax
from jax.experimental import pallas as pl
from jax.experimental.pallas import tpu as pltpu
from jax.experimental.pallas import tpu_sc as plsc

N = 10000
E = 320000
NC, NS, LANES = 2, 16, 16
NW = NC * NS
EPW = E // NW
CH = 400
NCHUNK = EPW // CH
PADW = 16
MW = 5
MAXW = N * MW
NEG = -3.0e38


def _sc_segment(ys_pad, tabs, src, dst, zeros_np):
    npass = len(tabs)
    mesh = plsc.VectorSubcoreMesh(core_axis_name="c", subcore_axis_name="s")

    @functools.partial(
        pl.kernel,
        out_type=(
            jax.ShapeDtypeStruct((N, PADW), jnp.float32),
            jax.ShapeDtypeStruct((N, PADW), jnp.float32),
            jax.ShapeDtypeStruct((npass * NW * MAXW,), jnp.float32),
        ),
        mesh=mesh,
        compiler_params=pltpu.CompilerParams(
            needs_layout_passes=False, use_tc_tiling_on_sc=False),
        scratch_types=[
            pltpu.VMEM((CH,), jnp.int32),
            pltpu.VMEM((CH,), jnp.int32),
            pltpu.VMEM((CH, PADW), jnp.float32),
            pltpu.VMEM((MAXW,), jnp.float32),
            pltpu.VMEM((MAXW,), jnp.float32),
            pltpu.VMEM_SHARED((N, PADW), jnp.float32),
            pltpu.SemaphoreType.DMA,
        ],
    )
    def k(ys_hbm, *args):
        tab_hbm = args[:npass]
        (src_hbm, dst_hbm, zeros_hbm, sum0_out, sum1_out, max_out,
         srcbuf, dstbuf, rows, table, maxacc, ssum, sem) = args[npass:]
        c = lax.axis_index("c")
        s = lax.axis_index("s")
        wid = s * NC + c
        iota16 = lax.iota(jnp.int32, LANES)

        @pl.when(s == 0)
        def _():
            pltpu.sync_copy(zeros_hbm, ssum)
        plsc.subcore_barrier()

        def _sum_chunk(kk, _):
            base = wid * EPW + kk * CH
            pltpu.sync_copy(src_hbm.at[pl.ds(base, CH)], srcbuf)
            pltpu.sync_copy(dst_hbm.at[pl.ds(base, CH)], dstbuf)
            pltpu.async_copy(ys_hbm.at[srcbuf], rows, sem).wait()
            pltpu.sync_copy(rows, ssum.at[dstbuf], add=True)
            return ()

        lax.fori_loop(0, NCHUNK, _sum_chunk, ())
        plsc.subcore_barrier()

        @pl.when(jnp.logical_and(s == 0, c == 0))
        def _():
            pltpu.sync_copy(ssum, sum0_out)

        @pl.when(jnp.logical_and(s == 0, c == 1))
        def _():
            pltpu.sync_copy(ssum, sum1_out)

        for p in range(npass):
            pltpu.sync_copy(tab_hbm[p], table)

            def _init(i, _):
                maxacc[pl.ds(i * LANES, LANES)] = jnp.full(
                    (LANES,), NEG, jnp.float32)
                return ()
            lax.fori_loop(0, MAXW // LANES, _init, ())

            def _chunk(kk, _):
                base = wid * EPW + kk * CH
                pltpu.sync_copy(src_hbm.at[pl.ds(base, CH)], srcbuf)
                pltpu.sync_copy(dst_hbm.at[pl.ds(base, CH)], dstbuf)

                def _vec(v, _):
                    e0 = v * LANES
                    dstv = dstbuf[pl.ds(e0, LANES)]
                    srcv = srcbuf[pl.ds(e0, LANES)]
                    aidx = [dstv * MW + j for j in range(MW)]
                    vals = [plsc.load_gather(table, [srcv * MW + j])
                            for j in range(MW)]

                    def _cond(m):
                        return jnp.any(m > 0)

                    def _body(m):
                        mb = m > 0
                        lost = jnp.zeros((LANES,), jnp.bool_)
                        for j in range(MW):
                            old = plsc.load_gather(maxacc, [aidx[j]], mask=mb)
                            new = jnp.maximum(old, vals[j])
                            plsc.store_scatter(maxacc, [aidx[j]], new, mask=mb)
                            back = plsc.load_gather(maxacc, [aidx[j]], mask=mb)
                            lost = jnp.logical_or(
                                lost, jnp.logical_and(mb, back < new))
                        return lost.astype(jnp.int32)

                    lax.while_loop(_cond, _body, jnp.ones((LANES,), jnp.int32))
                    return ()

                lax.fori_loop(0, CH // LANES, _vec, ())
                return ()

            lax.fori_loop(0, NCHUNK, _chunk, ())

            pltpu.sync_copy(
                maxacc, max_out.at[pl.ds((p * NW + wid) * MAXW, MAXW)])

    return k(ys_pad, *tabs, src, dst, zeros_np)


def _make_reduce(npass):
    def body(mp_ref, s0_ref, s1_ref, *out_refs):
        for p in range(npass):
            out_refs[p][...] = jnp.max(mp_ref[p * NW:(p + 1) * NW], axis=0)
        out_refs[npass][...] = s0_ref[...] + s1_ref[...]

    return body


def _reduce_max(mp, s0, s1, npass):
    mpf = mp.reshape(npass * NW, MAXW)
    outs = pl.pallas_call(
        _make_reduce(npass),
        out_shape=tuple(jax.ShapeDtypeStruct((MAXW,), jnp.float32)
                        for _ in range(npass))
        + (jax.ShapeDtypeStruct((N, PADW), jnp.float32),),
    )(mpf, s0, s1)
    return [o.reshape(N, MW) for o in outs[:npass]] + [outs[npass]]


def _pre_kernel(x_ref, wst_ref, wdt_ref, b_ref, ys_ref, ydb_ref, ta_ref, tb_ref):
    x = x_ref[...]
    ys = jnp.dot(x, wst_ref[...], preferred_element_type=jnp.float32)
    lane = lax.broadcasted_iota(jnp.int32, (N, PADW), 1)
    ys_ref[...] = jnp.where(lane == 10, 1.0, ys)
    ydb_ref[...] = jnp.dot(x, wdt_ref[...],
                           preferred_element_type=jnp.float32) + b_ref[...]
    ta_ref[...] = ys[:, 0:5]
    tb_ref[...] = ys[:, 5:10]


def _make_combine(h, hnext, head):
    npass = (h + MW - 1) // MW

    def body(sp_ref, *refs):
        mp_refs = refs[:npass]
        ydb_ref, g_ref, be_ref, *rest = refs[npass:]
        sp = sp_ref[...]
        cnt = sp[:, h:h + 1]
        S = sp[:, :h]
        mps = [mp_refs[p][...] for p in range(npass)]
        M = jnp.concatenate(mps, axis=1) if npass > 1 else mps[0]
        ydb = ydb_ref[...][:, :h]
        has = cnt > 0.0
        ssum = cnt * ydb + S
        mean = jnp.where(has, ydb + S / jnp.maximum(cnt, 1.0), 0.0)
        mx = jnp.where(has, ydb + M, 0.0)
        feat = jnp.concatenate([ssum, mean, mx], axis=1)
        mu = jnp.mean(feat, axis=0, keepdims=True)
        var = jnp.mean((feat - mu) ** 2, axis=0, keepdims=True)
        hm = jnp.maximum(
            (feat - mu) / jnp.sqrt(var + 1e-5) * g_ref[...] + be_ref[...], 0.0)
        if not head:
            wst_ref, wdt_ref, bn_ref, ys_ref, ydb2_ref, tab_ref = rest
            ysn = jnp.dot(hm, wst_ref[...], preferred_element_type=jnp.float32)
            lane = lax.broadcasted_iota(jnp.int32, (N, PADW), 1)
            ys_ref[...] = jnp.where(lane == hnext, 1.0, ysn)
            ydb2_ref[...] = jnp.dot(
                hm, wdt_ref[...], preferred_element_type=jnp.float32) + bn_ref[...]
            tab_ref[...] = ysn[:, 0:MW]
        else:
            wl1_ref, bl1_ref, g4_ref, be4_ref, wl2_ref, bl2_ref, out_ref = rest
            y = jnp.dot(hm, wl1_ref[...],
                        preferred_element_type=jnp.float32) + bl1_ref[...]
            mu2 = jnp.mean(y, axis=0, keepdims=True)
            var2 = jnp.mean((y - mu2) ** 2, axis=0, keepdims=True)
            y = (y - mu2) / jnp.sqrt(var2 + 1e-5) * g4_ref[...] + be4_ref[...]
            y = jnp.maximum(y, 0.0)
            out_ref[...] = jnp.dot(
                y, wl2_ref[...], preferred_element_type=jnp.float32) + bl2_ref[...]

    return body


def _split_pad(W, b):
    h, twof = W.shape
    f = twof // 2
    wdt = jnp.zeros((f, PADW), jnp.float32).at[:, :h].set(W[:, :f].T)
    wst = jnp.zeros((f, PADW), jnp.float32).at[:, :h].set(W[:, f:].T)
    bp = jnp.zeros((1, PADW), jnp.float32).at[0, :h].set(b)
    return wdt, wst, bp


def kernel(x, edge_index, W1, b1, g1, be1, W2, b2, g2, be2, W3, b3, g3, be3,
           Wl1, bl1, g4, be4, Wl2, bl2):
    src = edge_index[0]
    dst = edge_index[1]
    wdt1, wst1, bp1 = _split_pad(W1, b1)
    wdt2, wst2, bp2 = _split_pad(W2, b2)
    wdt3, wst3, bp3 = _split_pad(W3, b3)
    zeros_np = jnp.zeros((N, PADW), jnp.float32)

    ys1, ydb1, ta1, tb1 = pl.pallas_call(
        _pre_kernel,
        out_shape=(jax.ShapeDtypeStruct((N, PADW), jnp.float32),
                   jax.ShapeDtypeStruct((N, PADW), jnp.float32),
                   jax.ShapeDtypeStruct((N, MW), jnp.float32),
                   jax.ShapeDtypeStruct((N, MW), jnp.float32)),
    )(x, wst1, wdt1, bp1)

    s0, s1, mp1 = _sc_segment(
        ys1, [ta1.reshape(MAXW), tb1.reshape(MAXW)], src, dst, zeros_np)
    ma1, mb1, sc1 = _reduce_max(mp1, s0, s1, 2)

    ys2, ydb2, ta2 = pl.pallas_call(
        _make_combine(10, 5, False),
        out_shape=(jax.ShapeDtypeStruct((N, PADW), jnp.float32),
                   jax.ShapeDtypeStruct((N, PADW), jnp.float32),
                   jax.ShapeDtypeStruct((N, MW), jnp.float32)),
    )(sc1, ma1, mb1, ydb1, g1[None, :], be1[None, :], wst2, wdt2, bp2)

    s0, s1, mp2 = _sc_segment(ys2, [ta2.reshape(MAXW)], src, dst, zeros_np)
    ma2, sc2 = _reduce_max(mp2, s0, s1, 1)

    ys3, ydb3, ta3 = pl.pallas_call(
        _make_combine(5, 5, False),
        out_shape=(jax.ShapeDtypeStruct((N, PADW), jnp.float32),
                   jax.ShapeDtypeStruct((N, PADW), jnp.float32),
                   jax.ShapeDtypeStruct((N, MW), jnp.float32)),
    )(sc2, ma2, ydb2, g2[None, :], be2[None, :], wst3, wdt3, bp3)

    s0, s1, mp3 = _sc_segment(ys3, [ta3.reshape(MAXW)], src, dst, zeros_np)
    ma3, sc3 = _reduce_max(mp3, s0, s1, 1)

    out = pl.pallas_call(
        _make_combine(5, 0, True),
        out_shape=jax.ShapeDtypeStruct((N, 1), jnp.float32),
    )(sc3, ma3, ydb3, g3[None, :], be3[None, :],
      Wl1.T, bl1[None, :], g4[None, :], be4[None, :], Wl2.T, bl2[None, :])
    return out


# speedup vs baseline: 13.5513x; 1.1697x over previous
import functools

import jax
import jax.numpy as jnp
from jax import lax
from jax.experimental import pallas as pl
from jax.experimental.pallas import tpu as pltpu
from jax.experimental.pallas import tpu_sc as plsc

N = 10000
E = 320000
NC, NS, LANES = 2, 16, 16
NW = NC * NS
EPW = E // NW
CHM = 1000
NCHM = EPW // CHM
PADW = 16
MW = 5
MAXW = N * MW
NEG = -3.0e38


def _sc_segment(ys_pad, tabs, src, dst, zeros_np, rowpad, chs):
    npass = len(tabs)
    nchs = EPW // chs
    mesh = plsc.VectorSubcoreMesh(core_axis_name="c", subcore_axis_name="s")

    @functools.partial(
        pl.kernel,
        out_type=(
            jax.ShapeDtypeStruct((N, rowpad), jnp.float32),
            jax.ShapeDtypeStruct((N, rowpad), jnp.float32),
            jax.ShapeDtypeStruct((npass * NW * MAXW,), jnp.float32),
        ),
        mesh=mesh,
        compiler_params=pltpu.CompilerParams(
            needs_layout_passes=False, use_tc_tiling_on_sc=False),
        scratch_types=[
            pltpu.VMEM((chs,), jnp.int32),
            pltpu.VMEM((chs,), jnp.int32),
            pltpu.VMEM((CHM + LANES,), jnp.int32),
            pltpu.VMEM((CHM + LANES,), jnp.int32),
            pltpu.VMEM((chs, rowpad), jnp.float32),
            pltpu.VMEM((MAXW,), jnp.float32),
            pltpu.VMEM((MAXW,), jnp.float32),
            pltpu.VMEM((N,), jnp.int32),
            pltpu.VMEM_SHARED((N, rowpad), jnp.float32),
            pltpu.SemaphoreType.DMA,
        ],
    )
    def k(ys_hbm, *args):
        tab_hbm = args[:npass]
        (src_hbm, dst_hbm, zeros_hbm, sum0_out, sum1_out, max_out,
         srcbufs, dstbufs, srcbuf, dstbuf, rows, table, maxacc, tag, ssum,
         sem) = args[npass:]
        c = lax.axis_index("c")
        s = lax.axis_index("s")
        wid = s * NC + c
        iota16 = lax.iota(jnp.int32, LANES)

        @pl.when(s == 0)
        def _():
            pltpu.sync_copy(zeros_hbm, ssum)
        plsc.subcore_barrier()

        def _sum_chunk(kk, _):
            base = wid * EPW + kk * chs
            pltpu.sync_copy(src_hbm.at[pl.ds(base, chs)], srcbufs)
            pltpu.sync_copy(dst_hbm.at[pl.ds(base, chs)], dstbufs)
            pltpu.async_copy(ys_hbm.at[srcbufs], rows, sem).wait()
            pltpu.sync_copy(rows, ssum.at[dstbufs], add=True)
            return ()

        lax.fori_loop(0, nchs, _sum_chunk, ())
        plsc.subcore_barrier()

        @pl.when(jnp.logical_and(s == 0, c == 0))
        def _():
            pltpu.sync_copy(ssum, sum0_out)

        @pl.when(jnp.logical_and(s == 0, c == 1))
        def _():
            pltpu.sync_copy(ssum, sum1_out)

        for p in range(npass):
            pltpu.sync_copy(tab_hbm[p], table)

            def _init(i, _):
                maxacc[pl.ds(i * LANES, LANES)] = jnp.full(
                    (LANES,), NEG, jnp.float32)
                return ()
            lax.fori_loop(0, MAXW // LANES, _init, ())

            def _chunk(kk, _):
                base = wid * EPW + kk * CHM
                pltpu.sync_copy(src_hbm.at[pl.ds(base, CHM)],
                                srcbuf.at[pl.ds(0, CHM)])
                pltpu.sync_copy(dst_hbm.at[pl.ds(base, CHM)],
                                dstbuf.at[pl.ds(0, CHM)])

                def _vec(v, _):
                    e0 = v * LANES
                    mvec = (iota16 + e0) < CHM
                    dstv = jnp.where(mvec, dstbuf[pl.ds(e0, LANES)], 0)
                    srcv = jnp.where(mvec, srcbuf[pl.ds(e0, LANES)], 0)
                    aidx = [dstv * MW + j for j in range(MW)]
                    vals = [plsc.load_gather(table, [srcv * MW + j],
                                             mask=mvec)
                            for j in range(MW)]

                    plsc.store_scatter(tag, [dstv], iota16, mask=mvec)
                    back = plsc.load_gather(tag, [dstv], mask=mvec)

                    for j in range(MW):
                        old = plsc.load_gather(maxacc, [aidx[j]], mask=mvec)
                        plsc.store_scatter(
                            maxacc, [aidx[j]], jnp.maximum(old, vals[j]),
                            mask=mvec)

                    def _cond(m):
                        return jnp.any(m > 0)

                    def _body(m):
                        mb = m > 0
                        lost = jnp.zeros((LANES,), jnp.bool_)
                        for j in range(MW):
                            old = plsc.load_gather(maxacc, [aidx[j]], mask=mb)
                            new = jnp.maximum(old, vals[j])
                            plsc.store_scatter(maxacc, [aidx[j]], new, mask=mb)
                            bk = plsc.load_gather(maxacc, [aidx[j]], mask=mb)
                            lost = jnp.logical_or(
                                lost, jnp.logical_and(mb, bk < new))
                        return lost.astype(jnp.int32)

                    anydup = jnp.any(
                        jnp.logical_and(back != iota16, mvec)).astype(jnp.int32)
                    m1 = mvec.astype(jnp.int32) * anydup
                    lax.while_loop(_cond, _body, m1)
                    return ()

                lax.fori_loop(0, (CHM + LANES - 1) // LANES, _vec, ())
                return ()

            lax.fori_loop(0, NCHM, _chunk, ())

            pltpu.sync_copy(
                maxacc, max_out.at[pl.ds((p * NW + wid) * MAXW, MAXW)])

    return k(ys_pad, *tabs, src, dst, zeros_np)


def _make_reduce(npass):
    def body(mp_ref, s0_ref, s1_ref, *out_refs):
        for p in range(npass):
            out_refs[p][...] = jnp.max(mp_ref[p * NW:(p + 1) * NW], axis=0)
        out_refs[npass][...] = s0_ref[...] + s1_ref[...]

    return body


def _reduce_max(mp, s0, s1, npass):
    mpf = mp.reshape(npass * NW, MAXW)
    outs = pl.pallas_call(
        _make_reduce(npass),
        out_shape=tuple(jax.ShapeDtypeStruct((MAXW,), jnp.float32)
                        for _ in range(npass))
        + (jax.ShapeDtypeStruct(s0.shape, jnp.float32),),
    )(mpf, s0, s1)
    return [o.reshape(N, MW) for o in outs[:npass]] + [outs[npass]]


def _pre_kernel(x_ref, wst_ref, wdt_ref, b_ref, ys_ref, ydb_ref, ta_ref, tb_ref):
    x = x_ref[...]
    ys = jnp.dot(x, wst_ref[...], preferred_element_type=jnp.float32)
    lane = lax.broadcasted_iota(jnp.int32, ys.shape, 1)
    ys_ref[...] = jnp.where(lane == 10, 1.0, ys)
    ydb_ref[...] = jnp.dot(x, wdt_ref[...],
                           preferred_element_type=jnp.float32) + b_ref[...]
    ta_ref[...] = ys[:, 0:5]
    tb_ref[...] = ys[:, 5:10]


def _make_combine(h, hnext, head, padout=8):
    npass = (h + MW - 1) // MW

    def body(sp_ref, *refs):
        mp_refs = refs[:npass]
        ydb_ref, g_ref, be_ref, *rest = refs[npass:]
        sp = sp_ref[...]
        cnt = sp[:, h:h + 1]
        S = sp[:, :h]
        mps = [mp_refs[p][...] for p in range(npass)]
        M = jnp.concatenate(mps, axis=1) if npass > 1 else mps[0]
        ydb = ydb_ref[...][:, :h]
        has = cnt > 0.0
        ssum = cnt * ydb + S
        mean = jnp.where(has, ydb + S / jnp.maximum(cnt, 1.0), 0.0)
        mx = jnp.where(has, ydb + M, 0.0)
        feat = jnp.concatenate([ssum, mean, mx], axis=1)
        mu = jnp.mean(feat, axis=0, keepdims=True)
        var = jnp.mean((feat - mu) ** 2, axis=0, keepdims=True)
        hm = jnp.maximum(
            (feat - mu) / jnp.sqrt(var + 1e-5) * g_ref[...] + be_ref[...], 0.0)
        if not head:
            wst_ref, wdt_ref, bn_ref, ys_ref, ydb2_ref, tab_ref = rest
            ysn = jnp.dot(hm, wst_ref[...], preferred_element_type=jnp.float32)
            lane = lax.broadcasted_iota(jnp.int32, (N, padout), 1)
            ys_ref[...] = jnp.where(lane == hnext, 1.0, ysn)
            ydb2_ref[...] = jnp.dot(
                hm, wdt_ref[...], preferred_element_type=jnp.float32) + bn_ref[...]
            tab_ref[...] = ysn[:, 0:MW]
        else:
            wl1_ref, bl1_ref, g4_ref, be4_ref, wl2_ref, bl2_ref, out_ref = rest
            y = jnp.dot(hm, wl1_ref[...],
                        preferred_element_type=jnp.float32) + bl1_ref[...]
            mu2 = jnp.mean(y, axis=0, keepdims=True)
            var2 = jnp.mean((y - mu2) ** 2, axis=0, keepdims=True)
            y = (y - mu2) / jnp.sqrt(var2 + 1e-5) * g4_ref[...] + be4_ref[...]
            y = jnp.maximum(y, 0.0)
            out_ref[...] = jnp.dot(
                y, wl2_ref[...], preferred_element_type=jnp.float32) + bl2_ref[...]

    return body


def _split_pad(W, b, spad):
    h, twof = W.shape
    f = twof // 2
    wdt = jnp.zeros((f, PADW), jnp.float32).at[:, :h].set(W[:, :f].T)
    wst = jnp.zeros((f, spad), jnp.float32).at[:, :h].set(W[:, f:].T)
    bp = jnp.zeros((1, PADW), jnp.float32).at[0, :h].set(b)
    return wdt, wst, bp


def kernel(x, edge_index, W1, b1, g1, be1, W2, b2, g2, be2, W3, b3, g3, be3,
           Wl1, bl1, g4, be4, Wl2, bl2):
    src = edge_index[0]
    dst = edge_index[1]
    wdt1, wst1, bp1 = _split_pad(W1, b1, PADW)
    wdt2, wst2, bp2 = _split_pad(W2, b2, 8)
    wdt3, wst3, bp3 = _split_pad(W3, b3, 8)
    zeros16 = jnp.zeros((N, PADW), jnp.float32)
    zeros8 = jnp.zeros((N, 8), jnp.float32)

    ys1, ydb1, ta1, tb1 = pl.pallas_call(
        _pre_kernel,
        out_shape=(jax.ShapeDtypeStruct((N, PADW), jnp.float32),
                   jax.ShapeDtypeStruct((N, PADW), jnp.float32),
                   jax.ShapeDtypeStruct((N, MW), jnp.float32),
                   jax.ShapeDtypeStruct((N, MW), jnp.float32)),
    )(x, wst1, wdt1, bp1)

    s0, s1, mp1 = _sc_segment(
        ys1, [ta1.reshape(MAXW), tb1.reshape(MAXW)], src, dst, zeros16,
        PADW, 400)
    ma1, mb1, sc1 = _reduce_max(mp1, s0, s1, 2)

    ys2, ydb2, ta2 = pl.pallas_call(
        _make_combine(10, 5, False),
        out_shape=(jax.ShapeDtypeStruct((N, 8), jnp.float32),
                   jax.ShapeDtypeStruct((N, PADW), jnp.float32),
                   jax.ShapeDtypeStruct((N, MW), jnp.float32)),
    )(sc1, ma1, mb1, ydb1, g1[None, :], be1[None, :], wst2, wdt2, bp2)

    s0, s1, mp2 = _sc_segment(ys2, [ta2.reshape(MAXW)], src, dst, zeros8,
                              8, 1000)
    ma2, sc2 = _reduce_max(mp2, s0, s1, 1)

    ys3, ydb3, ta3 = pl.pallas_call(
        _make_combine(5, 5, False),
        out_shape=(jax.ShapeDtypeStruct((N, 8), jnp.float32),
                   jax.ShapeDtypeStruct((N, PADW), jnp.float32),
                   jax.ShapeDtypeStruct((N, MW), jnp.float32)),
    )(sc2, ma2, ydb2, g2[None, :], be2[None, :], wst3, wdt3, bp3)

    s0, s1, mp3 = _sc_segment(ys3, [ta3.reshape(MAXW)], src, dst, zeros8,
                              8, 1000)
    ma3, sc3 = _reduce_max(mp3, s0, s1, 1)

    out = pl.pallas_call(
        _make_combine(5, 0, True),
        out_shape=jax.ShapeDtypeStruct((N, 1), jnp.float32),
    )(sc3, ma3, ydb3, g3[None, :], be3[None, :],
      Wl1.T, bl1[None, :], g4[None, :], be4[None, :], Wl2.T, bl2[None, :])
    return out
